# baseline (speedup 1.0000x reference)
import functools

import jax
import jax.numpy as jnp
from jax import lax
from jax.experimental import pallas as pl
from jax.experimental.pallas import tpu as pltpu
from jax.experimental.pallas import tpu_sc as plsc

NC = 2
NS = 16
LANES = 16
CHUNK = 80
ZR = 128


def _sc_segment_accumulate(ent_lo, ent_hi, rel_lo, rel_hi,
                           src_r, rel_r, dst_r, norm_r, zrows):
  n_ent, d = ent_lo.shape
  n_reltab = rel_lo.shape[0]
  k_chunks, chunk = src_r.shape[2], src_r.shape[3]
  n_pad = ((n_ent + NS * ZR - 1) // (NS * ZR)) * (NS * ZR)
  rows_per_tile = n_pad // NS
  n_wcopy = rows_per_tile // ZR

  mesh = plsc.VectorSubcoreMesh(
      core_axis_name="c", subcore_axis_name="s", num_cores=NC, num_subcores=NS)

  @functools.partial(
      pl.kernel,
      out_type=jax.ShapeDtypeStruct((NC, 2, n_pad, d), jnp.float32),
      mesh=mesh,
      compiler_params=pltpu.CompilerParams(
          use_tc_tiling_on_sc=False, needs_layout_passes=False),
      scratch_types=[
          pltpu.VMEM((k_chunks, chunk), jnp.int32),
          pltpu.VMEM((k_chunks, chunk), jnp.int32),
          pltpu.VMEM((k_chunks, chunk), jnp.int32),
          pltpu.VMEM((k_chunks, chunk), jnp.float32),
          pltpu.VMEM((n_reltab, d), jnp.float32),
          pltpu.VMEM((chunk, d), jnp.float32),
          pltpu.VMEM((chunk, d), jnp.float32),
          pltpu.VMEM((chunk, d), jnp.float32),
          pltpu.VMEM((chunk, d), jnp.float32),
          pltpu.VMEM((ZR, d), jnp.float32),
          pltpu.VMEM_SHARED((n_pad, d), jnp.float32),
          pltpu.SemaphoreType.DMA,
          pltpu.SemaphoreType.DMA,
      ],
  )
  def sc_kernel(entl_hbm, enth_hbm, rell_hbm, relh_hbm,
                src_hbm, reli_hbm, dst_hbm, norm_hbm, zrows_hbm, out_hbm,
                src_v, rel_v, dst_v, norm_v, reltab, ent0, ent1,
                prod0, prod1, wbuf, acc_sh, sem_g, sem_s):
    c = lax.axis_index("c")
    s = lax.axis_index("s")
    row0 = s * rows_per_tile

    pltpu.sync_copy(src_hbm.at[c, s], src_v)
    pltpu.sync_copy(reli_hbm.at[c, s], rel_v)
    pltpu.sync_copy(dst_hbm.at[c, s], dst_v)
    pltpu.sync_copy(norm_hbm.at[c, s], norm_v)

    def compute_chunk(k, entb, prodb):
      def grp(j, carry):
        base = j * LANES
        ev = base + jnp.arange(LANES, dtype=jnp.int32)
        rel16 = rel_v[k, pl.ds(base, LANES)]
        n16 = norm_v[k, pl.ds(base, LANES)]
        for col in range(d):
          cf = jnp.full((LANES,), col, jnp.int32)
          e_c = plsc.load_gather(entb, [ev, cf])
          r_c = plsc.load_gather(reltab, [rel16, cf])
          plsc.store_scatter(prodb, [ev, cf], e_c * r_c * n16)
        return carry

      lax.fori_loop(0, chunk // LANES, grp, 0)

    for h, (e_hbm, r_hbm) in enumerate(
        ((entl_hbm, rell_hbm), (enth_hbm, relh_hbm))):
      pltpu.sync_copy(r_hbm, reltab)
      for i in range(n_wcopy):
        pltpu.sync_copy(zrows_hbm, acc_sh.at[pl.ds(row0 + i * ZR, ZR)])
      plsc.subcore_barrier()

      pltpu.async_copy(e_hbm.at[src_v.at[0]], ent0, sem_g)
      pltpu.async_copy(e_hbm.at[src_v.at[1]], ent1, sem_g)

      def body(i, carry):
        for b, (entb, prodb) in enumerate(((ent0, prod0), (ent1, prod1))):
          k = 2 * i + b
          pltpu.make_async_copy(e_hbm.at[src_v.at[k]], entb, sem_g).wait()
          compute_chunk(k, entb, prodb)

          @pl.when(k > 0)
          def _():
            pltpu.make_async_copy(
                prodb, acc_sh.at[dst_v.at[k]], sem_s).wait()

          pltpu.async_copy(prodb, acc_sh.at[dst_v.at[k]], sem_s, add=True)

          @pl.when(k + 2 < k_chunks)
          def _():
            pltpu.async_copy(e_hbm.at[src_v.at[k + 2]], entb, sem_g)
        return carry

      lax.fori_loop(0, k_chunks // 2, body, 0)

      kt = k_chunks - 1
      pltpu.make_async_copy(e_hbm.at[src_v.at[kt]], ent0, sem_g).wait()
      compute_chunk(kt, ent0, prod0)
      pltpu.make_async_copy(prod0, acc_sh.at[dst_v.at[kt]], sem_s).wait()
      pltpu.async_copy(prod0, acc_sh.at[dst_v.at[kt]], sem_s, add=True)
      pltpu.make_async_copy(prod0, acc_sh.at[dst_v.at[kt]], sem_s).wait()
      plsc.subcore_barrier()

      for i in range(n_wcopy):
        pltpu.sync_copy(acc_sh.at[pl.ds(row0 + i * ZR, ZR)], wbuf)
        pltpu.sync_copy(wbuf, out_hbm.at[c, h].at[pl.ds(row0 + i * ZR, ZR)])

  return sc_kernel(ent_lo, ent_hi, rel_lo, rel_hi,
                   src_r, rel_r, dst_r, norm_r, zrows)


def _tc1_body(a0l_ref, a0h_ref, a1l_ref, a1h_ref, ent_ref,
              inwl_ref, inwh_ref, outwl_ref, outwh_ref, loopw_ref,
              looprel_ref, bias_ref, relp_ref, wrel_ref,
              y_ref, ssum_ref, ssq_ref, rout_ref):
  i = pl.program_id(0)
  y = jnp.dot(a0l_ref[...], inwl_ref[...], preferred_element_type=jnp.float32)
  y = y + jnp.dot(a0h_ref[...], inwh_ref[...],
                  preferred_element_type=jnp.float32)
  y = y + jnp.dot(a1l_ref[...], outwl_ref[...],
                  preferred_element_type=jnp.float32)
  y = y + jnp.dot(a1h_ref[...], outwh_ref[...],
                  preferred_element_type=jnp.float32)
  y = y + jnp.dot(ent_ref[...] * looprel_ref[...], loopw_ref[...],
                  preferred_element_type=jnp.float32)
  y = y / 3.0 + bias_ref[...]
  y_ref[...] = y
  ps = jnp.sum(y, axis=0, keepdims=True)
  pq = jnp.sum(y * y, axis=0, keepdims=True)

  @pl.when(i == 0)
  def _():
    ssum_ref[...] = jnp.zeros_like(ssum_ref)
    ssq_ref[...] = jnp.zeros_like(ssq_ref)
    rout_ref[...] = jnp.dot(relp_ref[...], wrel_ref[...],
                            preferred_element_type=jnp.float32)

  ssum_ref[...] += jnp.broadcast_to(ps, ssum_ref.shape)
  ssq_ref[...] += jnp.broadcast_to(pq, ssq_ref.shape)


def _tc2_body(n_rows, y_ref, ssum_ref, ssq_ref, gamma_ref, beta_ref, x_ref):
  inv_n = 1.0 / n_rows
  mean = ssum_ref[0:1, :] * inv_n
  var = ssq_ref[0:1, :] * inv_n - mean * mean
  inv = lax.rsqrt(var + 1e-5)
  x_ref[...] = jnp.tanh(
      (y_ref[...] - mean) * inv * gamma_ref[...] + beta_ref[...])


def kernel(ent_emb, rel_emb, edge_index, relation, norm, triples,
           in_w, out_w, loop_w, w_rel, loop_rel, bias_p, bn_gamma, bn_beta):
  n_ent, d_in = ent_emb.shape
  n_rel = rel_emb.shape[0]
  d_out = in_w.shape[1]
  dh = d_in // 2
  e = edge_index.shape[1]
  per_tile = e // (NC * NS)
  k_chunks = per_tile // CHUNK

  shape4 = (NC, NS, k_chunks, CHUNK)
  src_r = edge_index[0].reshape(shape4)
  dst_r = edge_index[1].reshape(shape4)
  rel_r = relation.reshape(shape4).astype(jnp.int32)
  norm_r = norm.reshape(shape4)
  zrows = jnp.zeros((ZR, dh), jnp.float32)

  acc = _sc_segment_accumulate(
      ent_emb[:, :dh], ent_emb[:, dh:], rel_emb[:, :dh], rel_emb[:, dh:],
      src_r, rel_r, dst_r, norm_r, zrows)

  br = 2000
  nb = n_ent // br
  n_rel_pad = 240
  rel_pad = jnp.zeros((n_rel_pad, d_in), jnp.float32).at[:n_rel].set(rel_emb)
  looprel2 = loop_rel.reshape(1, d_in)
  bias2 = bias_p.reshape(1, d_out)
  gamma2 = bn_gamma.reshape(1, d_out)
  beta2 = bn_beta.reshape(1, d_out)

  acc_spec = lambda c, h: pl.BlockSpec(
      (1, 1, br, dh), lambda i, c=c, h=h: (c, h, i, 0))
  y, ssum, ssq, rout_pad = pl.pallas_call(
      _tc1_wrap,
      grid=(nb,),
      in_specs=[
          acc_spec(0, 0), acc_spec(0, 1), acc_spec(1, 0), acc_spec(1, 1),
          pl.BlockSpec((br, d_in), lambda i: (i, 0)),
          pl.BlockSpec((dh, d_out), lambda i: (0, 0)),
          pl.BlockSpec((dh, d_out), lambda i: (0, 0)),
          pl.BlockSpec((dh, d_out), lambda i: (0, 0)),
          pl.BlockSpec((dh, d_out), lambda i: (0, 0)),
          pl.BlockSpec((d_in, d_out), lambda i: (0, 0)),
          pl.BlockSpec((1, d_in), lambda i: (0, 0)),
          pl.BlockSpec((1, d_out), lambda i: (0, 0)),
          pl.BlockSpec((n_rel_pad, d_in), lambda i: (0, 0)),
          pl.BlockSpec((d_in, d_out), lambda i: (0, 0)),
      ],
      out_specs=[
          pl.BlockSpec((br, d_out), lambda i: (i, 0)),
          pl.BlockSpec((8, d_out), lambda i: (0, 0)),
          pl.BlockSpec((8, d_out), lambda i: (0, 0)),
          pl.BlockSpec((n_rel_pad, d_out), lambda i: (0, 0)),
      ],
      out_shape=[
          jax.ShapeDtypeStruct((n_ent, d_out), jnp.float32),
          jax.ShapeDtypeStruct((8, d_out), jnp.float32),
          jax.ShapeDtypeStruct((8, d_out), jnp.float32),
          jax.ShapeDtypeStruct((n_rel_pad, d_out), jnp.float32),
      ],
  )(acc, acc, acc, acc, ent_emb, in_w[:dh], in_w[dh:], out_w[:dh],
    out_w[dh:], loop_w, looprel2, bias2, rel_pad, w_rel)

  x = pl.pallas_call(
      functools.partial(_tc2_body, float(n_ent)),
      grid=(nb,),
      in_specs=[
          pl.BlockSpec((br, d_out), lambda i: (i, 0)),
          pl.BlockSpec((8, d_out), lambda i: (0, 0)),
          pl.BlockSpec((8, d_out), lambda i: (0, 0)),
          pl.BlockSpec((1, d_out), lambda i: (0, 0)),
          pl.BlockSpec((1, d_out), lambda i: (0, 0)),
      ],
      out_specs=pl.BlockSpec((br, d_out), lambda i: (i, 0)),
      out_shape=jax.ShapeDtypeStruct((n_ent, d_out), jnp.float32),
  )(y, ssum, ssq, gamma2, beta2)

  return (x, rout_pad[:n_rel])


def _tc1_wrap(a0l_ref, a0h_ref, a1l_ref, a1h_ref, *rest):
  _tc1_body(a0l_ref.at[0, 0], a0h_ref.at[0, 0], a1l_ref.at[0, 0],
            a1h_ref.at[0, 0], *rest)


# speedup vs baseline: 3.9291x; 3.9291x over previous
import functools

import jax
import jax.numpy as jnp
from jax import lax
from jax.experimental import pallas as pl
from jax.experimental.pallas import tpu as pltpu
from jax.experimental.pallas import tpu_sc as plsc

NC = 2
NS = 16
LANES = 16
CHUNK = 80
ZR = 128


def _sc_segment_accumulate(ent_lo, ent_hi, rel_lo, rel_hi,
                           src_r, rel_r, dst_r, norm_r, zrows):
  n_ent, d = ent_lo.shape
  n_reltab = rel_lo.shape[0]
  k_chunks, chunk = src_r.shape[2], src_r.shape[3]
  n_pad = ((n_ent + NS * ZR - 1) // (NS * ZR)) * (NS * ZR)
  rows_per_tile = n_pad // NS
  n_wcopy = rows_per_tile // ZR

  mesh = plsc.VectorSubcoreMesh(
      core_axis_name="c", subcore_axis_name="s", num_cores=NC, num_subcores=NS)

  @functools.partial(
      pl.kernel,
      out_type=jax.ShapeDtypeStruct((NC, 2, n_pad, d), jnp.float32),
      mesh=mesh,
      compiler_params=pltpu.CompilerParams(
          use_tc_tiling_on_sc=False, needs_layout_passes=False),
      scratch_types=[
          pltpu.VMEM((k_chunks, chunk), jnp.int32),
          pltpu.VMEM((k_chunks, chunk), jnp.int32),
          pltpu.VMEM((k_chunks, chunk), jnp.int32),
          pltpu.VMEM((k_chunks, chunk), jnp.float32),
          pltpu.VMEM((2 * chunk, d), jnp.float32),
          pltpu.VMEM((2 * chunk, d), jnp.float32),
          pltpu.VMEM((2 * chunk, d), jnp.float32),
          pltpu.VMEM((ZR, d), jnp.float32),
          pltpu.VMEM_SHARED((n_pad, d), jnp.float32),
          pltpu.SemaphoreType.DMA,
          pltpu.SemaphoreType.DMA,
      ],
  )
  def sc_kernel(entl_hbm, enth_hbm, rell_hbm, relh_hbm,
                src_hbm, reli_hbm, dst_hbm, norm_hbm, zrows_hbm, out_hbm,
                src_v, rel_v, dst_v, norm_v, ent2, rel2, prod2,
                wbuf, acc_sh, sem_g, sem_s):
    c = lax.axis_index("c")
    s = lax.axis_index("s")
    row0 = s * rows_per_tile

    pltpu.sync_copy(src_hbm.at[c, s], src_v)
    pltpu.sync_copy(reli_hbm.at[c, s], rel_v)
    pltpu.sync_copy(dst_hbm.at[c, s], dst_v)
    pltpu.sync_copy(norm_hbm.at[c, s], norm_v)

    def compute_chunk(k, boff):
      def grp(j, carry):
        base = j * LANES
        norm16 = norm_v[k, pl.ds(base, LANES)]
        for l in range(LANES):
          e = boff + base + l
          lv = jnp.full((LANES,), l, jnp.int32)
          nv = norm16.at[lv].get(mode="promise_in_bounds")
          for dd in range(d // LANES):
            sl = pl.ds(dd * LANES, LANES)
            prod2[e, sl] = ent2[e, sl] * rel2[e, sl] * nv
        return carry

      lax.fori_loop(0, chunk // LANES, grp, 0)

    for h, (e_hbm, r_hbm) in enumerate(
        ((entl_hbm, rell_hbm), (enth_hbm, relh_hbm))):
      for i in range(n_wcopy):
        pltpu.sync_copy(zrows_hbm, acc_sh.at[pl.ds(row0 + i * ZR, ZR)])
      plsc.subcore_barrier()

      kt = k_chunks - 1
      half0 = ent2.at[pl.ds(0, chunk)]
      pltpu.async_copy(e_hbm.at[src_v.at[0]], half0, sem_g)
      pltpu.async_copy(r_hbm.at[rel_v.at[0]], rel2.at[pl.ds(0, chunk)], sem_g)
      pltpu.async_copy(e_hbm.at[src_v.at[1]], ent2.at[pl.ds(chunk, chunk)],
                       sem_g)
      pltpu.async_copy(r_hbm.at[rel_v.at[1]], rel2.at[pl.ds(chunk, chunk)],
                       sem_g)
      pltpu.make_async_copy(e_hbm.at[src_v.at[0]], half0, sem_g).wait()
      pltpu.make_async_copy(e_hbm.at[src_v.at[0]], half0, sem_g).wait()
      compute_chunk(0, 0)
      pltpu.async_copy(prod2.at[pl.ds(0, chunk)], acc_sh.at[dst_v.at[0]],
                       sem_s, add=True)
      pltpu.async_copy(e_hbm.at[src_v.at[2]], half0, sem_g)
      pltpu.async_copy(r_hbm.at[rel_v.at[2]], rel2.at[pl.ds(0, chunk)], sem_g)

      def body(k, carry):
        boff = (k % 2) * chunk
        eslc = ent2.at[pl.ds(boff, chunk)]
        rslc = rel2.at[pl.ds(boff, chunk)]
        pslc = prod2.at[pl.ds(boff, chunk)]
        pltpu.make_async_copy(e_hbm.at[src_v.at[k]], eslc, sem_g).wait()
        pltpu.make_async_copy(e_hbm.at[src_v.at[k]], eslc, sem_g).wait()
        compute_chunk(k, boff)
        pltpu.make_async_copy(pslc, acc_sh.at[dst_v.at[k]], sem_s).wait()
        pltpu.async_copy(pslc, acc_sh.at[dst_v.at[k]], sem_s, add=True)
        knext = jnp.minimum(k + 2, kt)
        pltpu.async_copy(e_hbm.at[src_v.at[knext]], eslc, sem_g)
        pltpu.async_copy(r_hbm.at[rel_v.at[knext]], rslc, sem_g)
        return carry

      lax.fori_loop(1, k_chunks, body, 0)

      pltpu.make_async_copy(prod2.at[pl.ds(0, chunk)],
                            acc_sh.at[dst_v.at[kt]], sem_s).wait()
      for _ in range(4):
        pltpu.make_async_copy(e_hbm.at[src_v.at[kt]], half0, sem_g).wait()
      plsc.subcore_barrier()

      for i in range(n_wcopy):
        pltpu.sync_copy(acc_sh.at[pl.ds(row0 + i * ZR, ZR)], wbuf)
        pltpu.sync_copy(wbuf, out_hbm.at[c, h].at[pl.ds(row0 + i * ZR, ZR)])

  return sc_kernel(ent_lo, ent_hi, rel_lo, rel_hi,
                   src_r, rel_r, dst_r, norm_r, zrows)


def _tc1_body(a0l_ref, a0h_ref, a1l_ref, a1h_ref, ent_ref,
              inwl_ref, inwh_ref, outwl_ref, outwh_ref, loopw_ref,
              looprel_ref, bias_ref, relp_ref, wrel_ref,
              y_ref, ssum_ref, ssq_ref, rout_ref):
  i = pl.program_id(0)
  y = jnp.dot(a0l_ref[...], inwl_ref[...], preferred_element_type=jnp.float32)
  y = y + jnp.dot(a0h_ref[...], inwh_ref[...],
                  preferred_element_type=jnp.float32)
  y = y + jnp.dot(a1l_ref[...], outwl_ref[...],
                  preferred_element_type=jnp.float32)
  y = y + jnp.dot(a1h_ref[...], outwh_ref[...],
                  preferred_element_type=jnp.float32)
  y = y + jnp.dot(ent_ref[...] * looprel_ref[...], loopw_ref[...],
                  preferred_element_type=jnp.float32)
  y = y / 3.0 + bias_ref[...]
  y_ref[...] = y
  ps = jnp.sum(y, axis=0, keepdims=True)
  pq = jnp.sum(y * y, axis=0, keepdims=True)

  @pl.when(i == 0)
  def _():
    ssum_ref[...] = jnp.zeros_like(ssum_ref)
    ssq_ref[...] = jnp.zeros_like(ssq_ref)
    rout_ref[...] = jnp.dot(relp_ref[...], wrel_ref[...],
                            preferred_element_type=jnp.float32)

  ssum_ref[...] += jnp.broadcast_to(ps, ssum_ref.shape)
  ssq_ref[...] += jnp.broadcast_to(pq, ssq_ref.shape)


def _tc2_body(n_rows, y_ref, ssum_ref, ssq_ref, gamma_ref, beta_ref, x_ref):
  inv_n = 1.0 / n_rows
  mean = ssum_ref[0:1, :] * inv_n
  var = ssq_ref[0:1, :] * inv_n - mean * mean
  inv = lax.rsqrt(var + 1e-5)
  x_ref[...] = jnp.tanh(
      (y_ref[...] - mean) * inv * gamma_ref[...] + beta_ref[...])


def kernel(ent_emb, rel_emb, edge_index, relation, norm, triples,
           in_w, out_w, loop_w, w_rel, loop_rel, bias_p, bn_gamma, bn_beta):
  n_ent, d_in = ent_emb.shape
  n_rel = rel_emb.shape[0]
  d_out = in_w.shape[1]
  dh = d_in // 2
  e = edge_index.shape[1]
  per_tile = e // (NC * NS)
  k_chunks = per_tile // CHUNK

  shape4 = (NC, NS, k_chunks, CHUNK)
  src_r = edge_index[0].reshape(shape4)
  dst_r = edge_index[1].reshape(shape4)
  rel_r = relation.reshape(shape4).astype(jnp.int32)
  norm_r = norm.reshape(shape4)
  zrows = jnp.zeros((ZR, dh), jnp.float32)

  acc = _sc_segment_accumulate(
      ent_emb[:, :dh], ent_emb[:, dh:], rel_emb[:, :dh], rel_emb[:, dh:],
      src_r, rel_r, dst_r, norm_r, zrows)

  br = 2000
  nb = n_ent // br
  n_rel_pad = 240
  rel_pad = jnp.zeros((n_rel_pad, d_in), jnp.float32).at[:n_rel].set(rel_emb)
  looprel2 = loop_rel.reshape(1, d_in)
  bias2 = bias_p.reshape(1, d_out)
  gamma2 = bn_gamma.reshape(1, d_out)
  beta2 = bn_beta.reshape(1, d_out)

  acc_spec = lambda c, h: pl.BlockSpec(
      (1, 1, br, dh), lambda i, c=c, h=h: (c, h, i, 0))
  y, ssum, ssq, rout_pad = pl.pallas_call(
      _tc1_wrap,
      grid=(nb,),
      in_specs=[
          acc_spec(0, 0), acc_spec(0, 1), acc_spec(1, 0), acc_spec(1, 1),
          pl.BlockSpec((br, d_in), lambda i: (i, 0)),
          pl.BlockSpec((dh, d_out), lambda i: (0, 0)),
          pl.BlockSpec((dh, d_out), lambda i: (0, 0)),
          pl.BlockSpec((dh, d_out), lambda i: (0, 0)),
          pl.BlockSpec((dh, d_out), lambda i: (0, 0)),
          pl.BlockSpec((d_in, d_out), lambda i: (0, 0)),
          pl.BlockSpec((1, d_in), lambda i: (0, 0)),
          pl.BlockSpec((1, d_out), lambda i: (0, 0)),
          pl.BlockSpec((n_rel_pad, d_in), lambda i: (0, 0)),
          pl.BlockSpec((d_in, d_out), lambda i: (0, 0)),
      ],
      out_specs=[
          pl.BlockSpec((br, d_out), lambda i: (i, 0)),
          pl.BlockSpec((8, d_out), lambda i: (0, 0)),
          pl.BlockSpec((8, d_out), lambda i: (0, 0)),
          pl.BlockSpec((n_rel_pad, d_out), lambda i: (0, 0)),
      ],
      out_shape=[
          jax.ShapeDtypeStruct((n_ent, d_out), jnp.float32),
          jax.ShapeDtypeStruct((8, d_out), jnp.float32),
          jax.ShapeDtypeStruct((8, d_out), jnp.float32),
          jax.ShapeDtypeStruct((n_rel_pad, d_out), jnp.float32),
      ],
  )(acc, acc, acc, acc, ent_emb, in_w[:dh], in_w[dh:], out_w[:dh],
    out_w[dh:], loop_w, looprel2, bias2, rel_pad, w_rel)

  x = pl.pallas_call(
      functools.partial(_tc2_body, float(n_ent)),
      grid=(nb,),
      in_specs=[
          pl.BlockSpec((br, d_out), lambda i: (i, 0)),
          pl.BlockSpec((8, d_out), lambda i: (0, 0)),
          pl.BlockSpec((8, d_out), lambda i: (0, 0)),
          pl.BlockSpec((1, d_out), lambda i: (0, 0)),
          pl.BlockSpec((1, d_out), lambda i: (0, 0)),
      ],
      out_specs=pl.BlockSpec((br, d_out), lambda i: (i, 0)),
      out_shape=jax.ShapeDtypeStruct((n_ent, d_out), jnp.float32),
  )(y, ssum, ssq, gamma2, beta2)

  return (x, rout_pad[:n_rel])


def _tc1_wrap(a0l_ref, a0h_ref, a1l_ref, a1h_ref, *rest):
  _tc1_body(a0l_ref.at[0, 0], a0h_ref.at[0, 0], a1l_ref.at[0, 0],
            a1h_ref.at[0, 0], *rest)


# speedup vs baseline: 3.9375x; 1.0021x over previous
import functools

import jax
import jax.numpy as jnp
from jax import lax
from jax.experimental import pallas as pl
from jax.experimental.pallas import tpu as pltpu
from jax.experimental.pallas import tpu_sc as plsc

NC = 2
NS = 16
LANES = 16
CHUNK = 80
ZR = 128


def _sc_segment_accumulate(ent_lo, ent_hi, relsc,
                           src_r, rel_r, dst_r, norm_r, zrows):
  n_ent, d = ent_lo.shape
  n_reltab = relsc.shape[1]
  rel_rows = n_reltab // NS
  k_chunks, chunk = src_r.shape[2], src_r.shape[3]
  n_pad = ((n_ent + NS * ZR - 1) // (NS * ZR)) * (NS * ZR)
  rows_per_tile = n_pad // NS
  n_wcopy = rows_per_tile // ZR

  mesh = plsc.VectorSubcoreMesh(
      core_axis_name="c", subcore_axis_name="s", num_cores=NC, num_subcores=NS)

  @functools.partial(
      pl.kernel,
      out_type=jax.ShapeDtypeStruct((NC, 2, n_pad, d), jnp.float32),
      mesh=mesh,
      compiler_params=pltpu.CompilerParams(
          use_tc_tiling_on_sc=False, needs_layout_passes=False),
      scratch_types=[
          pltpu.VMEM((k_chunks, chunk), jnp.int32),
          pltpu.VMEM((k_chunks, chunk), jnp.int32),
          pltpu.VMEM((k_chunks, chunk), jnp.int32),
          pltpu.VMEM((k_chunks, chunk), jnp.float32),
          pltpu.VMEM((2 * chunk, d), jnp.float32),
          pltpu.VMEM((2 * chunk, d), jnp.float32),
          pltpu.VMEM((2 * chunk, d), jnp.float32),
          pltpu.VMEM((ZR, d), jnp.float32),
          pltpu.VMEM_SHARED((n_pad, d), jnp.float32),
          pltpu.VMEM_SHARED((2, n_reltab, d), jnp.float32),
          pltpu.SemaphoreType.DMA,
          pltpu.SemaphoreType.DMA,
          pltpu.SemaphoreType.DMA,
      ],
  )
  def sc_kernel(entl_hbm, enth_hbm, relsc_hbm,
                src_hbm, reli_hbm, dst_hbm, norm_hbm, zrows_hbm, out_hbm,
                src_v, rel_v, dst_v, norm_v, ent2, rel2, prod2,
                wbuf, acc_sh, rel_sh, sem_ge, sem_gr, sem_s):
    c = lax.axis_index("c")
    s = lax.axis_index("s")
    row0 = s * rows_per_tile

    pltpu.sync_copy(src_hbm.at[c, s], src_v)
    pltpu.sync_copy(reli_hbm.at[c, s], rel_v)
    pltpu.sync_copy(dst_hbm.at[c, s], dst_v)
    pltpu.sync_copy(norm_hbm.at[c, s], norm_v)
    rsl = pl.ds(s * rel_rows, rel_rows)
    pltpu.sync_copy(relsc_hbm.at[0].at[rsl], rel_sh.at[0].at[rsl])
    pltpu.sync_copy(relsc_hbm.at[1].at[rsl], rel_sh.at[1].at[rsl])

    def compute_chunk(k, boff):
      def grp(j, carry):
        base = j * LANES
        norm16 = norm_v[k, pl.ds(base, LANES)]
        for l in range(LANES):
          e = boff + base + l
          lv = jnp.full((LANES,), l, jnp.int32)
          nv = norm16.at[lv].get(mode="promise_in_bounds")
          for dd in range(d // LANES):
            sl = pl.ds(dd * LANES, LANES)
            prod2[e, sl] = ent2[e, sl] * rel2[e, sl] * nv
        return carry

      lax.fori_loop(0, chunk // LANES, grp, 0)

    for h, e_hbm in enumerate((entl_hbm, enth_hbm)):
      r_src = rel_sh.at[h]
      for i in range(n_wcopy):
        pltpu.sync_copy(zrows_hbm, acc_sh.at[pl.ds(row0 + i * ZR, ZR)])
      plsc.subcore_barrier()

      kt = k_chunks - 1
      half0 = ent2.at[pl.ds(0, chunk)]
      rhalf0 = rel2.at[pl.ds(0, chunk)]
      pltpu.async_copy(e_hbm.at[src_v.at[0]], half0, sem_ge)
      pltpu.async_copy(r_src.at[rel_v.at[0]], rhalf0, sem_gr)
      pltpu.async_copy(e_hbm.at[src_v.at[1]], ent2.at[pl.ds(chunk, chunk)],
                       sem_ge)
      pltpu.async_copy(r_src.at[rel_v.at[1]], rel2.at[pl.ds(chunk, chunk)],
                       sem_gr)
      pltpu.make_async_copy(e_hbm.at[src_v.at[0]], half0, sem_ge).wait()
      pltpu.make_async_copy(r_src.at[rel_v.at[0]], rhalf0, sem_gr).wait()
      compute_chunk(0, 0)
      pltpu.async_copy(prod2.at[pl.ds(0, chunk)], acc_sh.at[dst_v.at[0]],
                       sem_s, add=True)
      pltpu.async_copy(e_hbm.at[src_v.at[2]], half0, sem_ge)
      pltpu.async_copy(r_src.at[rel_v.at[2]], rhalf0, sem_gr)

      def body(k, carry):
        boff = (k % 2) * chunk
        eslc = ent2.at[pl.ds(boff, chunk)]
        rslc = rel2.at[pl.ds(boff, chunk)]
        pslc = prod2.at[pl.ds(boff, chunk)]
        pltpu.make_async_copy(e_hbm.at[src_v.at[k]], eslc, sem_ge).wait()
        pltpu.make_async_copy(r_src.at[rel_v.at[k]], rslc, sem_gr).wait()
        compute_chunk(k, boff)
        pltpu.make_async_copy(pslc, acc_sh.at[dst_v.at[k]], sem_s).wait()
        pltpu.async_copy(pslc, acc_sh.at[dst_v.at[k]], sem_s, add=True)
        knext = jnp.minimum(k + 2, kt)
        pltpu.async_copy(e_hbm.at[src_v.at[knext]], eslc, sem_ge)
        pltpu.async_copy(r_src.at[rel_v.at[knext]], rslc, sem_gr)
        return carry

      lax.fori_loop(1, k_chunks, body, 0)

      pltpu.make_async_copy(prod2.at[pl.ds(0, chunk)],
                            acc_sh.at[dst_v.at[kt]], sem_s).wait()
      for _ in range(2):
        pltpu.make_async_copy(e_hbm.at[src_v.at[kt]], half0, sem_ge).wait()
        pltpu.make_async_copy(r_src.at[rel_v.at[kt]], rhalf0, sem_gr).wait()
      plsc.subcore_barrier()

      for i in range(n_wcopy):
        pltpu.sync_copy(acc_sh.at[pl.ds(row0 + i * ZR, ZR)], wbuf)
        pltpu.sync_copy(wbuf, out_hbm.at[c, h].at[pl.ds(row0 + i * ZR, ZR)])

  return sc_kernel(ent_lo, ent_hi, relsc,
                   src_r, rel_r, dst_r, norm_r, zrows)


def _tc1_body(a0l_ref, a0h_ref, a1l_ref, a1h_ref, ent_ref,
              inwl_ref, inwh_ref, outwl_ref, outwh_ref, loopw_ref,
              looprel_ref, bias_ref, relp_ref, wrel_ref,
              y_ref, ssum_ref, ssq_ref, rout_ref):
  i = pl.program_id(0)
  y = jnp.dot(a0l_ref[...], inwl_ref[...], preferred_element_type=jnp.float32)
  y = y + jnp.dot(a0h_ref[...], inwh_ref[...],
                  preferred_element_type=jnp.float32)
  y = y + jnp.dot(a1l_ref[...], outwl_ref[...],
                  preferred_element_type=jnp.float32)
  y = y + jnp.dot(a1h_ref[...], outwh_ref[...],
                  preferred_element_type=jnp.float32)
  y = y + jnp.dot(ent_ref[...] * looprel_ref[...], loopw_ref[...],
                  preferred_element_type=jnp.float32)
  y = y / 3.0 + bias_ref[...]
  y_ref[...] = y
  ps = jnp.sum(y, axis=0, keepdims=True)
  pq = jnp.sum(y * y, axis=0, keepdims=True)

  @pl.when(i == 0)
  def _():
    ssum_ref[...] = jnp.zeros_like(ssum_ref)
    ssq_ref[...] = jnp.zeros_like(ssq_ref)
    rout_ref[...] = jnp.dot(relp_ref[...], wrel_ref[...],
                            preferred_element_type=jnp.float32)

  ssum_ref[...] += jnp.broadcast_to(ps, ssum_ref.shape)
  ssq_ref[...] += jnp.broadcast_to(pq, ssq_ref.shape)


def _tc2_body(n_rows, y_ref, ssum_ref, ssq_ref, gamma_ref, beta_ref, x_ref):
  inv_n = 1.0 / n_rows
  mean = ssum_ref[0:1, :] * inv_n
  var = ssq_ref[0:1, :] * inv_n - mean * mean
  inv = lax.rsqrt(var + 1e-5)
  x_ref[...] = jnp.tanh(
      (y_ref[...] - mean) * inv * gamma_ref[...] + beta_ref[...])


def kernel(ent_emb, rel_emb, edge_index, relation, norm, triples,
           in_w, out_w, loop_w, w_rel, loop_rel, bias_p, bn_gamma, bn_beta):
  n_ent, d_in = ent_emb.shape
  n_rel = rel_emb.shape[0]
  d_out = in_w.shape[1]
  dh = d_in // 2
  e = edge_index.shape[1]
  per_tile = e // (NC * NS)
  k_chunks = per_tile // CHUNK

  shape4 = (NC, NS, k_chunks, CHUNK)
  src_r = edge_index[0].reshape(shape4)
  dst_r = edge_index[1].reshape(shape4)
  rel_r = relation.reshape(shape4).astype(jnp.int32)
  norm_r = norm.reshape(shape4)
  zrows = jnp.zeros((ZR, dh), jnp.float32)
  n_rel_sc = 240
  relsc = jnp.zeros((2, n_rel_sc, dh), jnp.float32)
  relsc = relsc.at[0, :n_rel].set(rel_emb[:, :dh])
  relsc = relsc.at[1, :n_rel].set(rel_emb[:, dh:])

  acc = _sc_segment_accumulate(
      ent_emb[:, :dh], ent_emb[:, dh:], relsc,
      src_r, rel_r, dst_r, norm_r, zrows)

  br = 2000
  nb = n_ent // br
  n_rel_pad = 240
  rel_pad = jnp.zeros((n_rel_pad, d_in), jnp.float32).at[:n_rel].set(rel_emb)
  looprel2 = loop_rel.reshape(1, d_in)
  bias2 = bias_p.reshape(1, d_out)
  gamma2 = bn_gamma.reshape(1, d_out)
  beta2 = bn_beta.reshape(1, d_out)

  acc_spec = lambda c, h: pl.BlockSpec(
      (1, 1, br, dh), lambda i, c=c, h=h: (c, h, i, 0))
  y, ssum, ssq, rout_pad = pl.pallas_call(
      _tc1_wrap,
      grid=(nb,),
      in_specs=[
          acc_spec(0, 0), acc_spec(0, 1), acc_spec(1, 0), acc_spec(1, 1),
          pl.BlockSpec((br, d_in), lambda i: (i, 0)),
          pl.BlockSpec((dh, d_out), lambda i: (0, 0)),
          pl.BlockSpec((dh, d_out), lambda i: (0, 0)),
          pl.BlockSpec((dh, d_out), lambda i: (0, 0)),
          pl.BlockSpec((dh, d_out), lambda i: (0, 0)),
          pl.BlockSpec((d_in, d_out), lambda i: (0, 0)),
          pl.BlockSpec((1, d_in), lambda i: (0, 0)),
          pl.BlockSpec((1, d_out), lambda i: (0, 0)),
          pl.BlockSpec((n_rel_pad, d_in), lambda i: (0, 0)),
          pl.BlockSpec((d_in, d_out), lambda i: (0, 0)),
      ],
      out_specs=[
          pl.BlockSpec((br, d_out), lambda i: (i, 0)),
          pl.BlockSpec((8, d_out), lambda i: (0, 0)),
          pl.BlockSpec((8, d_out), lambda i: (0, 0)),
          pl.BlockSpec((n_rel_pad, d_out), lambda i: (0, 0)),
      ],
      out_shape=[
          jax.ShapeDtypeStruct((n_ent, d_out), jnp.float32),
          jax.ShapeDtypeStruct((8, d_out), jnp.float32),
          jax.ShapeDtypeStruct((8, d_out), jnp.float32),
          jax.ShapeDtypeStruct((n_rel_pad, d_out), jnp.float32),
      ],
  )(acc, acc, acc, acc, ent_emb, in_w[:dh], in_w[dh:], out_w[:dh],
    out_w[dh:], loop_w, looprel2, bias2, rel_pad, w_rel)

  x = pl.pallas_call(
      functools.partial(_tc2_body, float(n_ent)),
      grid=(nb,),
      in_specs=[
          pl.BlockSpec((br, d_out), lambda i: (i, 0)),
          pl.BlockSpec((8, d_out), lambda i: (0, 0)),
          pl.BlockSpec((8, d_out), lambda i: (0, 0)),
          pl.BlockSpec((1, d_out), lambda i: (0, 0)),
          pl.BlockSpec((1, d_out), lambda i: (0, 0)),
      ],
      out_specs=pl.BlockSpec((br, d_out), lambda i: (i, 0)),
      out_shape=jax.ShapeDtypeStruct((n_ent, d_out), jnp.float32),
  )(y, ssum, ssq, gamma2, beta2)

  return (x, rout_pad[:n_rel])


def _tc1_wrap(a0l_ref, a0h_ref, a1l_ref, a1h_ref, *rest):
  _tc1_body(a0l_ref.at[0, 0], a0h_ref.at[0, 0], a1l_ref.at[0, 0],
            a1h_ref.at[0, 0], *rest)


# speedup vs baseline: 7.4623x; 1.8952x over previous
import functools

import jax
import jax.numpy as jnp
from jax import lax
from jax.experimental import pallas as pl
from jax.experimental.pallas import tpu as pltpu
from jax.experimental.pallas import tpu_sc as plsc

NC = 2
NS = 16
LANES = 16
CHUNK = 80
ZR = 128


def _sc_segment_accumulate(ent_lo, ent_hi, relsc,
                           src_r, rel_r, dst_r, norm_r, zrows):
  n_ent, d = ent_lo.shape
  n_reltab = relsc.shape[1]
  rel_rows = n_reltab // NS
  k_chunks, chunk = src_r.shape[2], src_r.shape[3]
  n_pad = ((n_ent + NS * ZR - 1) // (NS * ZR)) * (NS * ZR)
  rows_per_tile = n_pad // NS
  n_wcopy = rows_per_tile // ZR

  mesh = plsc.VectorSubcoreMesh(
      core_axis_name="c", subcore_axis_name="s", num_cores=NC, num_subcores=NS)

  @functools.partial(
      pl.kernel,
      out_type=jax.ShapeDtypeStruct((NC, 2, n_pad, d), jnp.float32),
      mesh=mesh,
      compiler_params=pltpu.CompilerParams(
          use_tc_tiling_on_sc=False, needs_layout_passes=False),
      scratch_types=[
          pltpu.VMEM((k_chunks, chunk), jnp.int32),
          pltpu.VMEM((k_chunks, chunk), jnp.int32),
          pltpu.VMEM((k_chunks, chunk), jnp.int32),
          pltpu.VMEM((k_chunks, chunk), jnp.float32),
          pltpu.VMEM((2 * chunk, d), jnp.float32),
          pltpu.VMEM((2 * chunk, d), jnp.float32),
          pltpu.VMEM((2 * chunk, d), jnp.float32),
          pltpu.VMEM((ZR, d), jnp.float32),
          pltpu.VMEM_SHARED((n_pad, d), jnp.float32),
          pltpu.VMEM_SHARED((2, n_reltab, d), jnp.float32),
          pltpu.SemaphoreType.DMA,
          pltpu.SemaphoreType.DMA,
          pltpu.SemaphoreType.DMA,
      ],
  )
  def sc_kernel(entl_hbm, enth_hbm, relsc_hbm,
                src_hbm, reli_hbm, dst_hbm, norm_hbm, zrows_hbm, out_hbm,
                src_v, rel_v, dst_v, norm_v, ent2, rel2, prod2,
                wbuf, acc_sh, rel_sh, sem_ge, sem_gr, sem_s):
    c = lax.axis_index("c")
    s = lax.axis_index("s")
    row0 = s * rows_per_tile

    pltpu.sync_copy(src_hbm.at[c, s], src_v)
    pltpu.sync_copy(reli_hbm.at[c, s], rel_v)
    pltpu.sync_copy(dst_hbm.at[c, s], dst_v)
    pltpu.sync_copy(norm_hbm.at[c, s], norm_v)
    rsl = pl.ds(s * rel_rows, rel_rows)
    pltpu.sync_copy(relsc_hbm.at[0].at[rsl], rel_sh.at[0].at[rsl])
    pltpu.sync_copy(relsc_hbm.at[1].at[rsl], rel_sh.at[1].at[rsl])

    def compute_chunk(k, boff):
      for j in range(chunk // LANES):
        base = j * LANES
        norm16 = norm_v[k, pl.ds(base, LANES)]
        for l in range(LANES):
          e = boff + base + l
          nv = norm16[l]
          for dd in range(d // LANES):
            sl = pl.ds(dd * LANES, LANES)
            prod2[e, sl] = ent2[e, sl] * rel2[e, sl] * nv

    for h, e_hbm in enumerate((entl_hbm, enth_hbm)):
      r_src = rel_sh.at[h]
      for i in range(n_wcopy):
        pltpu.sync_copy(zrows_hbm, acc_sh.at[pl.ds(row0 + i * ZR, ZR)])
      plsc.subcore_barrier()

      kt = k_chunks - 1
      half0 = ent2.at[pl.ds(0, chunk)]
      rhalf0 = rel2.at[pl.ds(0, chunk)]
      pltpu.async_copy(e_hbm.at[src_v.at[0]], half0, sem_ge)
      pltpu.async_copy(r_src.at[rel_v.at[0]], rhalf0, sem_gr)
      pltpu.async_copy(e_hbm.at[src_v.at[1]], ent2.at[pl.ds(chunk, chunk)],
                       sem_ge)
      pltpu.async_copy(r_src.at[rel_v.at[1]], rel2.at[pl.ds(chunk, chunk)],
                       sem_gr)
      pltpu.make_async_copy(e_hbm.at[src_v.at[0]], half0, sem_ge).wait()
      pltpu.make_async_copy(r_src.at[rel_v.at[0]], rhalf0, sem_gr).wait()
      compute_chunk(0, 0)
      pltpu.async_copy(prod2.at[pl.ds(0, chunk)], acc_sh.at[dst_v.at[0]],
                       sem_s, add=True)
      pltpu.async_copy(e_hbm.at[src_v.at[2]], half0, sem_ge)
      pltpu.async_copy(r_src.at[rel_v.at[2]], rhalf0, sem_gr)

      def body(k, carry):
        boff = (k % 2) * chunk
        eslc = ent2.at[pl.ds(boff, chunk)]
        rslc = rel2.at[pl.ds(boff, chunk)]
        pslc = prod2.at[pl.ds(boff, chunk)]
        pltpu.make_async_copy(e_hbm.at[src_v.at[k]], eslc, sem_ge).wait()
        pltpu.make_async_copy(r_src.at[rel_v.at[k]], rslc, sem_gr).wait()
        compute_chunk(k, boff)
        pltpu.make_async_copy(pslc, acc_sh.at[dst_v.at[k]], sem_s).wait()
        pltpu.async_copy(pslc, acc_sh.at[dst_v.at[k]], sem_s, add=True)
        knext = jnp.minimum(k + 2, kt)
        pltpu.async_copy(e_hbm.at[src_v.at[knext]], eslc, sem_ge)
        pltpu.async_copy(r_src.at[rel_v.at[knext]], rslc, sem_gr)
        return carry

      lax.fori_loop(1, k_chunks, body, 0)

      pltpu.make_async_copy(prod2.at[pl.ds(0, chunk)],
                            acc_sh.at[dst_v.at[kt]], sem_s).wait()
      for _ in range(2):
        pltpu.make_async_copy(e_hbm.at[src_v.at[kt]], half0, sem_ge).wait()
        pltpu.make_async_copy(r_src.at[rel_v.at[kt]], rhalf0, sem_gr).wait()
      plsc.subcore_barrier()

      for i in range(n_wcopy):
        pltpu.sync_copy(acc_sh.at[pl.ds(row0 + i * ZR, ZR)], wbuf)
        pltpu.sync_copy(wbuf, out_hbm.at[c, h].at[pl.ds(row0 + i * ZR, ZR)])

  return sc_kernel(ent_lo, ent_hi, relsc,
                   src_r, rel_r, dst_r, norm_r, zrows)


def _tc1_body(a0l_ref, a0h_ref, a1l_ref, a1h_ref, ent_ref,
              inwl_ref, inwh_ref, outwl_ref, outwh_ref, loopw_ref,
              looprel_ref, bias_ref, relp_ref, wrel_ref,
              y_ref, ssum_ref, ssq_ref, rout_ref):
  i = pl.program_id(0)
  y = jnp.dot(a0l_ref[...], inwl_ref[...], preferred_element_type=jnp.float32)
  y = y + jnp.dot(a0h_ref[...], inwh_ref[...],
                  preferred_element_type=jnp.float32)
  y = y + jnp.dot(a1l_ref[...], outwl_ref[...],
                  preferred_element_type=jnp.float32)
  y = y + jnp.dot(a1h_ref[...], outwh_ref[...],
                  preferred_element_type=jnp.float32)
  y = y + jnp.dot(ent_ref[...] * looprel_ref[...], loopw_ref[...],
                  preferred_element_type=jnp.float32)
  y = y / 3.0 + bias_ref[...]
  y_ref[...] = y
  ps = jnp.sum(y, axis=0, keepdims=True)
  pq = jnp.sum(y * y, axis=0, keepdims=True)

  @pl.when(i == 0)
  def _():
    ssum_ref[...] = jnp.zeros_like(ssum_ref)
    ssq_ref[...] = jnp.zeros_like(ssq_ref)
    rout_ref[...] = jnp.dot(relp_ref[...], wrel_ref[...],
                            preferred_element_type=jnp.float32)

  ssum_ref[...] += jnp.broadcast_to(ps, ssum_ref.shape)
  ssq_ref[...] += jnp.broadcast_to(pq, ssq_ref.shape)


def _tc2_body(n_rows, y_ref, ssum_ref, ssq_ref, gamma_ref, beta_ref, x_ref):
  inv_n = 1.0 / n_rows
  mean = ssum_ref[0:1, :] * inv_n
  var = ssq_ref[0:1, :] * inv_n - mean * mean
  inv = lax.rsqrt(var + 1e-5)
  x_ref[...] = jnp.tanh(
      (y_ref[...] - mean) * inv * gamma_ref[...] + beta_ref[...])


def kernel(ent_emb, rel_emb, edge_index, relation, norm, triples,
           in_w, out_w, loop_w, w_rel, loop_rel, bias_p, bn_gamma, bn_beta):
  n_ent, d_in = ent_emb.shape
  n_rel = rel_emb.shape[0]
  d_out = in_w.shape[1]
  dh = d_in // 2
  e = edge_index.shape[1]
  per_tile = e // (NC * NS)
  k_chunks = per_tile // CHUNK

  shape4 = (NC, NS, k_chunks, CHUNK)
  src_r = edge_index[0].reshape(shape4)
  dst_r = edge_index[1].reshape(shape4)
  rel_r = relation.reshape(shape4).astype(jnp.int32)
  norm_r = norm.reshape(shape4)
  zrows = jnp.zeros((ZR, dh), jnp.float32)
  n_rel_sc = 240
  relsc = jnp.zeros((2, n_rel_sc, dh), jnp.float32)
  relsc = relsc.at[0, :n_rel].set(rel_emb[:, :dh])
  relsc = relsc.at[1, :n_rel].set(rel_emb[:, dh:])

  acc = _sc_segment_accumulate(
      ent_emb[:, :dh], ent_emb[:, dh:], relsc,
      src_r, rel_r, dst_r, norm_r, zrows)

  br = 2000
  nb = n_ent // br
  n_rel_pad = 240
  rel_pad = jnp.zeros((n_rel_pad, d_in), jnp.float32).at[:n_rel].set(rel_emb)
  looprel2 = loop_rel.reshape(1, d_in)
  bias2 = bias_p.reshape(1, d_out)
  gamma2 = bn_gamma.reshape(1, d_out)
  beta2 = bn_beta.reshape(1, d_out)

  acc_spec = lambda c, h: pl.BlockSpec(
      (1, 1, br, dh), lambda i, c=c, h=h: (c, h, i, 0))
  y, ssum, ssq, rout_pad = pl.pallas_call(
      _tc1_wrap,
      grid=(nb,),
      in_specs=[
          acc_spec(0, 0), acc_spec(0, 1), acc_spec(1, 0), acc_spec(1, 1),
          pl.BlockSpec((br, d_in), lambda i: (i, 0)),
          pl.BlockSpec((dh, d_out), lambda i: (0, 0)),
          pl.BlockSpec((dh, d_out), lambda i: (0, 0)),
          pl.BlockSpec((dh, d_out), lambda i: (0, 0)),
          pl.BlockSpec((dh, d_out), lambda i: (0, 0)),
          pl.BlockSpec((d_in, d_out), lambda i: (0, 0)),
          pl.BlockSpec((1, d_in), lambda i: (0, 0)),
          pl.BlockSpec((1, d_out), lambda i: (0, 0)),
          pl.BlockSpec((n_rel_pad, d_in), lambda i: (0, 0)),
          pl.BlockSpec((d_in, d_out), lambda i: (0, 0)),
      ],
      out_specs=[
          pl.BlockSpec((br, d_out), lambda i: (i, 0)),
          pl.BlockSpec((8, d_out), lambda i: (0, 0)),
          pl.BlockSpec((8, d_out), lambda i: (0, 0)),
          pl.BlockSpec((n_rel_pad, d_out), lambda i: (0, 0)),
      ],
      out_shape=[
          jax.ShapeDtypeStruct((n_ent, d_out), jnp.float32),
          jax.ShapeDtypeStruct((8, d_out), jnp.float32),
          jax.ShapeDtypeStruct((8, d_out), jnp.float32),
          jax.ShapeDtypeStruct((n_rel_pad, d_out), jnp.float32),
      ],
  )(acc, acc, acc, acc, ent_emb, in_w[:dh], in_w[dh:], out_w[:dh],
    out_w[dh:], loop_w, looprel2, bias2, rel_pad, w_rel)

  x = pl.pallas_call(
      functools.partial(_tc2_body, float(n_ent)),
      grid=(nb,),
      in_specs=[
          pl.BlockSpec((br, d_out), lambda i: (i, 0)),
          pl.BlockSpec((8, d_out), lambda i: (0, 0)),
          pl.BlockSpec((8, d_out), lambda i: (0, 0)),
          pl.BlockSpec((1, d_out), lambda i: (0, 0)),
          pl.BlockSpec((1, d_out), lambda i: (0, 0)),
      ],
      out_specs=pl.BlockSpec((br, d_out), lambda i: (i, 0)),
      out_shape=jax.ShapeDtypeStruct((n_ent, d_out), jnp.float32),
  )(y, ssum, ssq, gamma2, beta2)

  return (x, rout_pad[:n_rel])


def _tc1_wrap(a0l_ref, a0h_ref, a1l_ref, a1h_ref, *rest):
  _tc1_body(a0l_ref.at[0, 0], a0h_ref.at[0, 0], a1l_ref.at[0, 0],
            a1h_ref.at[0, 0], *rest)


# speedup vs baseline: 7.6296x; 1.0224x over previous
import functools

import jax
import jax.numpy as jnp
from jax import lax
from jax.experimental import pallas as pl
from jax.experimental.pallas import tpu as pltpu
from jax.experimental.pallas import tpu_sc as plsc

NC = 2
NS = 16
LANES = 16
CHUNK = 80
ZR = 128


def _sc_segment_accumulate(ent_lo, ent_hi, relsc,
                           src_r, rel_r, dst_r, norm_r, zrows):
  n_ent, d = ent_lo.shape
  n_reltab = relsc.shape[1]
  rel_rows = n_reltab // NS
  k_chunks, chunk = src_r.shape[2], src_r.shape[3]
  n_pad = ((n_ent + NS * ZR - 1) // (NS * ZR)) * (NS * ZR)
  rows_per_tile = n_pad // NS
  n_wcopy = rows_per_tile // ZR

  mesh = plsc.VectorSubcoreMesh(
      core_axis_name="c", subcore_axis_name="s", num_cores=NC, num_subcores=NS)

  @functools.partial(
      pl.kernel,
      out_type=jax.ShapeDtypeStruct((NC, 2, n_pad, d), jnp.float32),
      mesh=mesh,
      compiler_params=pltpu.CompilerParams(
          use_tc_tiling_on_sc=False, needs_layout_passes=False),
      scratch_types=[
          pltpu.VMEM((k_chunks, chunk), jnp.int32),
          pltpu.VMEM((k_chunks, chunk), jnp.int32),
          pltpu.VMEM((k_chunks, chunk), jnp.int32),
          pltpu.VMEM((k_chunks, chunk), jnp.float32),
          pltpu.VMEM((2 * chunk, d), jnp.float32),
          pltpu.VMEM((2 * chunk, d), jnp.float32),
          pltpu.VMEM((2 * chunk, d), jnp.float32),
          pltpu.VMEM((ZR, d), jnp.float32),
          pltpu.VMEM_SHARED((n_pad, d), jnp.float32),
          pltpu.VMEM_SHARED((2, n_reltab, d), jnp.float32),
          pltpu.SemaphoreType.DMA,
          pltpu.SemaphoreType.DMA,
          pltpu.SemaphoreType.DMA,
      ],
  )
  def sc_kernel(entl_hbm, enth_hbm, relsc_hbm,
                src_hbm, reli_hbm, dst_hbm, norm_hbm, zrows_hbm, out_hbm,
                src_v, rel_v, dst_v, norm_v, ent2, rel2, prod2,
                wbuf, acc_sh, rel_sh, sem_ge, sem_gr, sem_s):
    c = lax.axis_index("c")
    s = lax.axis_index("s")
    row0 = s * rows_per_tile

    pltpu.sync_copy(src_hbm.at[c, s], src_v)
    pltpu.sync_copy(reli_hbm.at[c, s], rel_v)
    pltpu.sync_copy(dst_hbm.at[c, s], dst_v)
    pltpu.sync_copy(norm_hbm.at[c, s], norm_v)
    rsl = pl.ds(s * rel_rows, rel_rows)
    pltpu.sync_copy(relsc_hbm.at[0].at[rsl], rel_sh.at[0].at[rsl])
    pltpu.sync_copy(relsc_hbm.at[1].at[rsl], rel_sh.at[1].at[rsl])

    def compute_chunk(k, boff):
      for base in range(0, chunk, LANES):
        lo = min(base, chunk - LANES)
        norm16 = norm_v[k, pl.ds(lo, LANES)]
        for l in range(base - lo, min(LANES, chunk - base) + (base - lo)):
          e = boff + lo + l
          nv = norm16[l]
          for dd in range(d // LANES):
            sl = pl.ds(dd * LANES, LANES)
            prod2[e, sl] = ent2[e, sl] * rel2[e, sl] * nv

    for h, e_hbm in enumerate((entl_hbm, enth_hbm)):
      r_src = rel_sh.at[h]
      for i in range(n_wcopy):
        pltpu.sync_copy(zrows_hbm, acc_sh.at[pl.ds(row0 + i * ZR, ZR)])
      plsc.subcore_barrier()

      kt = k_chunks - 1
      half0 = ent2.at[pl.ds(0, chunk)]
      rhalf0 = rel2.at[pl.ds(0, chunk)]
      pltpu.async_copy(e_hbm.at[src_v.at[0]], half0, sem_ge)
      pltpu.async_copy(r_src.at[rel_v.at[0]], rhalf0, sem_gr)
      pltpu.async_copy(e_hbm.at[src_v.at[1]], ent2.at[pl.ds(chunk, chunk)],
                       sem_ge)
      pltpu.async_copy(r_src.at[rel_v.at[1]], rel2.at[pl.ds(chunk, chunk)],
                       sem_gr)
      pltpu.make_async_copy(e_hbm.at[src_v.at[0]], half0, sem_ge).wait()
      pltpu.make_async_copy(r_src.at[rel_v.at[0]], rhalf0, sem_gr).wait()
      compute_chunk(0, 0)
      pltpu.async_copy(prod2.at[pl.ds(0, chunk)], acc_sh.at[dst_v.at[0]],
                       sem_s, add=True)
      pltpu.async_copy(e_hbm.at[src_v.at[2]], half0, sem_ge)
      pltpu.async_copy(r_src.at[rel_v.at[2]], rhalf0, sem_gr)

      def body(k, carry):
        boff = (k % 2) * chunk
        eslc = ent2.at[pl.ds(boff, chunk)]
        rslc = rel2.at[pl.ds(boff, chunk)]
        pslc = prod2.at[pl.ds(boff, chunk)]
        pltpu.make_async_copy(e_hbm.at[src_v.at[k]], eslc, sem_ge).wait()
        pltpu.make_async_copy(r_src.at[rel_v.at[k]], rslc, sem_gr).wait()
        compute_chunk(k, boff)
        pltpu.make_async_copy(pslc, acc_sh.at[dst_v.at[k]], sem_s).wait()
        pltpu.async_copy(pslc, acc_sh.at[dst_v.at[k]], sem_s, add=True)
        knext = jnp.minimum(k + 2, kt)
        pltpu.async_copy(e_hbm.at[src_v.at[knext]], eslc, sem_ge)
        pltpu.async_copy(r_src.at[rel_v.at[knext]], rslc, sem_gr)
        return carry

      lax.fori_loop(1, k_chunks, body, 0)

      pltpu.make_async_copy(prod2.at[pl.ds(0, chunk)],
                            acc_sh.at[dst_v.at[kt]], sem_s).wait()
      for _ in range(2):
        pltpu.make_async_copy(e_hbm.at[src_v.at[kt]], half0, sem_ge).wait()
        pltpu.make_async_copy(r_src.at[rel_v.at[kt]], rhalf0, sem_gr).wait()
      plsc.subcore_barrier()

      for i in range(n_wcopy):
        pltpu.sync_copy(acc_sh.at[pl.ds(row0 + i * ZR, ZR)], wbuf)
        pltpu.sync_copy(wbuf, out_hbm.at[c, h].at[pl.ds(row0 + i * ZR, ZR)])

  return sc_kernel(ent_lo, ent_hi, relsc,
                   src_r, rel_r, dst_r, norm_r, zrows)


def _tc_fused(n_rows, br, a0l_ref, a0h_ref, a1l_ref, a1h_ref, ent_ref,
              inwl_ref, inwh_ref, outwl_ref, outwh_ref, loopw_ref,
              looprel_ref, bias_ref, relp_ref, wrel_ref,
              gamma_ref, beta_ref,
              x_ref, rout_ref, y_scr, ssum_ref, ssq_ref):
  p = pl.program_id(0)
  i = pl.program_id(1)
  rows = pl.ds(i * br, br)

  @pl.when(p == 0)
  def _():
    y = jnp.dot(a0l_ref[0, 0], inwl_ref[...],
                preferred_element_type=jnp.float32)
    y = y + jnp.dot(a0h_ref[0, 0], inwh_ref[...],
                    preferred_element_type=jnp.float32)
    y = y + jnp.dot(a1l_ref[0, 0], outwl_ref[...],
                    preferred_element_type=jnp.float32)
    y = y + jnp.dot(a1h_ref[0, 0], outwh_ref[...],
                    preferred_element_type=jnp.float32)
    y = y + jnp.dot(ent_ref[...] * looprel_ref[...], loopw_ref[...],
                    preferred_element_type=jnp.float32)
    y = y / 3.0 + bias_ref[...]
    y_scr[rows, :] = y
    ps = jnp.sum(y, axis=0, keepdims=True)
    pq = jnp.sum(y * y, axis=0, keepdims=True)

    @pl.when(i == 0)
    def _():
      ssum_ref[...] = jnp.zeros_like(ssum_ref)
      ssq_ref[...] = jnp.zeros_like(ssq_ref)
      rout_ref[...] = jnp.dot(relp_ref[...], wrel_ref[...],
                              preferred_element_type=jnp.float32)

    ssum_ref[...] += jnp.broadcast_to(ps, ssum_ref.shape)
    ssq_ref[...] += jnp.broadcast_to(pq, ssq_ref.shape)

  @pl.when(p == 1)
  def _():
    inv_n = 1.0 / n_rows
    mean = ssum_ref[0:1, :] * inv_n
    var = ssq_ref[0:1, :] * inv_n - mean * mean
    inv = lax.rsqrt(var + 1e-5)
    x_ref[...] = jnp.tanh(
        (y_scr[rows, :] - mean) * inv * gamma_ref[...] + beta_ref[...])


def kernel(ent_emb, rel_emb, edge_index, relation, norm, triples,
           in_w, out_w, loop_w, w_rel, loop_rel, bias_p, bn_gamma, bn_beta):
  n_ent, d_in = ent_emb.shape
  n_rel = rel_emb.shape[0]
  d_out = in_w.shape[1]
  dh = d_in // 2
  e = edge_index.shape[1]
  per_tile = e // (NC * NS)
  k_chunks = per_tile // CHUNK

  shape4 = (NC, NS, k_chunks, CHUNK)
  src_r = edge_index[0].reshape(shape4)
  dst_r = edge_index[1].reshape(shape4)
  rel_r = relation.reshape(shape4).astype(jnp.int32)
  norm_r = norm.reshape(shape4)
  zrows = jnp.zeros((ZR, dh), jnp.float32)
  n_rel_sc = 240
  relsc = jnp.zeros((2, n_rel_sc, dh), jnp.float32)
  relsc = relsc.at[0, :n_rel].set(rel_emb[:, :dh])
  relsc = relsc.at[1, :n_rel].set(rel_emb[:, dh:])

  acc = _sc_segment_accumulate(
      ent_emb[:, :dh], ent_emb[:, dh:], relsc,
      src_r, rel_r, dst_r, norm_r, zrows)

  br = 2000
  nb = n_ent // br
  n_rel_pad = 240
  rel_pad = jnp.zeros((n_rel_pad, d_in), jnp.float32).at[:n_rel].set(rel_emb)
  looprel2 = loop_rel.reshape(1, d_in)
  bias2 = bias_p.reshape(1, d_out)
  gamma2 = bn_gamma.reshape(1, d_out)
  beta2 = bn_beta.reshape(1, d_out)

  acc_spec = lambda c, h: pl.BlockSpec(
      (1, 1, br, dh),
      lambda p, i, c=c, h=h: (c, h, jnp.where(p == 0, i, 0), 0))
  const = lambda p, i: (0, 0)
  x, rout_pad = pl.pallas_call(
      functools.partial(_tc_fused, float(n_ent), br),
      grid=(2, nb),
      in_specs=[
          acc_spec(0, 0), acc_spec(0, 1), acc_spec(1, 0), acc_spec(1, 1),
          pl.BlockSpec((br, d_in),
                       lambda p, i: (jnp.where(p == 0, i, 0), 0)),
          pl.BlockSpec((dh, d_out), const),
          pl.BlockSpec((dh, d_out), const),
          pl.BlockSpec((dh, d_out), const),
          pl.BlockSpec((dh, d_out), const),
          pl.BlockSpec((d_in, d_out), const),
          pl.BlockSpec((1, d_in), const),
          pl.BlockSpec((1, d_out), const),
          pl.BlockSpec((n_rel_pad, d_in), const),
          pl.BlockSpec((d_in, d_out), const),
          pl.BlockSpec((1, d_out), const),
          pl.BlockSpec((1, d_out), const),
      ],
      out_specs=[
          pl.BlockSpec((br, d_out),
                       lambda p, i: (jnp.where(p == 1, i, 0), 0)),
          pl.BlockSpec((n_rel_pad, d_out), const),
      ],
      out_shape=[
          jax.ShapeDtypeStruct((n_ent, d_out), jnp.float32),
          jax.ShapeDtypeStruct((n_rel_pad, d_out), jnp.float32),
      ],
      scratch_shapes=[
          pltpu.VMEM((n_ent, d_out), jnp.float32),
          pltpu.VMEM((8, d_out), jnp.float32),
          pltpu.VMEM((8, d_out), jnp.float32),
      ],
  )(acc, acc, acc, acc, ent_emb, in_w[:dh], in_w[dh:], out_w[:dh],
    out_w[dh:], loop_w, looprel2, bias2, rel_pad, w_rel, gamma2, beta2)

  return (x, rout_pad[:n_rel])
